# baseline (speedup 1.0000x reference)
import functools
import math

import jax
import jax.numpy as jnp
from jax import lax
from jax.experimental import pallas as pl
from jax.experimental.pallas import tpu as pltpu
from jax.experimental.pallas import tpu_sc as plsc

N = 10000
NP = 10240
E = 320000
NC = 2
NS = 16
NW = NC * NS
CH = 128
K = 80
EPW = K * CH
EP = NW * EPW
RPT = NP // NS
L = 16
DUMMY = N
INV_SQRT1P = 1.0 / math.sqrt(1.0 + 1e-5)

_mesh = plsc.VectorSubcoreMesh(core_axis_name="c", subcore_axis_name="s")
_sc_params = pltpu.CompilerParams(use_tc_tiling_on_sc=False,
                                  needs_layout_passes=False)


NBUF = 8
LAG = NBUF // 2


def _make_agg(F):
    @functools.partial(
        pl.kernel,
        out_type=jax.ShapeDtypeStruct((NP, NC * F), jnp.float32),
        mesh=_mesh,
        compiler_params=_sc_params,
        scratch_types=[
            pltpu.VMEM((K, CH), jnp.int32),
            pltpu.VMEM((K, CH), jnp.int32),
            pltpu.VMEM((NBUF, 2, CH, F), jnp.float32),
            pltpu.VMEM_SHARED((NP, F), jnp.float32),
            pltpu.VMEM_SHARED((NP, F), jnp.float32),
        ] + [pltpu.SemaphoreType.DMA] * (2 * NBUF),
    )
    def agg(table_hbm, src_hbm, dst_hbm, zeros_hbm, out_hbm,
            src_v, dst_v, rows_v, acc_sh, table_sh, *sems):
        gsem = sems[:NBUF]
        ssem = sems[NBUF:]
        c = lax.axis_index("c")
        s = lax.axis_index("s")
        w = c * NS + s

        pltpu.sync_copy(src_hbm.at[w], src_v)
        pltpu.sync_copy(dst_hbm.at[w], dst_v)
        pltpu.sync_copy(table_hbm.at[pl.ds(s * RPT, RPT)],
                        table_sh.at[pl.ds(s * RPT, RPT)])
        pltpu.sync_copy(zeros_hbm.at[pl.ds(s * RPT, RPT)],
                        acc_sh.at[pl.ds(s * RPT, RPT)])
        plsc.subcore_barrier()

        K2 = K // 2

        def gather(j):
            b = j % NBUF
            for h in range(2):
                pltpu.async_copy(table_sh.at[src_v.at[2 * j + h]],
                                 rows_v.at[b, h], gsem[b])

        def gather_wait(j):
            b = j % NBUF
            for h in range(2):
                pltpu.make_async_copy(table_sh.at[src_v.at[2 * j + h]],
                                      rows_v.at[b, h], gsem[b]).wait()

        def scatter(j):
            b = j % NBUF
            for h in range(2):
                pltpu.async_copy(rows_v.at[b, h],
                                 acc_sh.at[dst_v.at[2 * j + h]],
                                 ssem[b], add=True)

        def scatter_wait(j):
            b = j % NBUF
            for h in range(2):
                pltpu.make_async_copy(rows_v.at[b, h],
                                      acc_sh.at[dst_v.at[2 * j + h]],
                                      ssem[b]).wait()

        for j in range(NBUF):
            gather(j)
        for j in range(K2):
            gather_wait(j)
            scatter(j)
            jj = j - LAG
            if jj >= 0 and jj + NBUF < K2:
                scatter_wait(jj)
                gather(jj + NBUF)
        for jj in range(max(0, K2 - NBUF), K2):
            scatter_wait(jj)

        plsc.subcore_barrier()
        pltpu.sync_copy(acc_sh.at[pl.ds(s * RPT, RPT)],
                        out_hbm.at[pl.ds(s * RPT, RPT), pl.ds(c * F, F)])

    return agg


_agg32 = _make_agg(32)


def _make_agg1(with_table):
    scratch = [
        pltpu.VMEM((EPW,), jnp.int32),
        pltpu.VMEM((NP,), jnp.float32),
    ]
    if with_table:
        scratch = [pltpu.VMEM((EPW,), jnp.int32)] + scratch
        scratch.append(pltpu.VMEM((NP,), jnp.float32))

    @functools.partial(
        pl.kernel,
        out_type=jax.ShapeDtypeStruct((NW, NP), jnp.float32),
        mesh=_mesh,
        compiler_params=_sc_params,
        scratch_types=scratch,
    )
    def agg1(*refs):
        if with_table:
            (table_hbm, src_hbm, dst_hbm, out_hbm,
             src_v, dst_v, acc_v, table_v) = refs
        else:
            dst_hbm, out_hbm, dst_v, acc_v = refs
        c = lax.axis_index("c")
        s = lax.axis_index("s")
        w = c * NS + s

        pltpu.sync_copy(dst_hbm.at[w], dst_v)
        if with_table:
            pltpu.sync_copy(src_hbm.at[w], src_v)
            pltpu.sync_copy(table_hbm, table_v)

        zeros16 = jnp.zeros((L,), jnp.float32)

        @plsc.parallel_loop(0, NP, step=L, unroll=8)
        def _(i):
            acc_v[pl.ds(i, L)] = zeros16

        ones16 = jnp.ones((L,), jnp.float32)

        @plsc.parallel_loop(0, EPW, step=L, unroll=8)
        def _(i):
            d16 = dst_v[pl.ds(i, L)]
            if with_table:
                s16 = src_v[pl.ds(i, L)]
                v16 = plsc.load_gather(table_v, [s16])
            else:
                v16 = ones16
            plsc.addupdate_scatter(acc_v, [d16], v16)

        pltpu.sync_copy(acc_v, out_hbm.at[w])

    return agg1


_agg1 = _make_agg1(True)
_degree = _make_agg1(False)


def _tc1a_body(x_ref, w1_ref, h_ref):
    h_ref[...] = jnp.dot(x_ref[...], w1_ref[...],
                         preferred_element_type=jnp.float32)


_tc1a = pl.pallas_call(
    _tc1a_body, out_shape=jax.ShapeDtypeStruct((NP, 32), jnp.float32))


def _tc1b_body(h_ref, degp_ref, hs_ref, dinv_ref):
    deg = jnp.sum(degp_ref[...], axis=1, keepdims=True) + 1.0
    dinv = lax.rsqrt(deg)
    dinv_ref[...] = dinv
    hs_ref[...] = h_ref[...] * dinv


_tc1b = pl.pallas_call(
    _tc1b_body,
    out_shape=(jax.ShapeDtypeStruct((NP, 32), jnp.float32),
               jax.ShapeDtypeStruct((NP, 1), jnp.float32)),
)


def _make_tc_mid(Fout):
    def body(aggp_ref, hs_ref, dinv_ref, b_ref, g_ref, be_ref, w_ref, out_ref):
        dinv = dinv_ref[...]
        aggp = aggp_ref[...]
        agg = (aggp[:, :32] + aggp[:, 32:] + hs_ref[...]) * dinv + b_ref[...]
        hin = jnp.maximum(agg * INV_SQRT1P * g_ref[...] + be_ref[...], 0.0)
        h = jnp.dot(hin, w_ref[...], preferred_element_type=jnp.float32)
        out_ref[...] = h * dinv

    return pl.pallas_call(
        body, out_shape=jax.ShapeDtypeStruct((NP, Fout), jnp.float32))


_tc2 = _make_tc_mid(32)
_tc3 = _make_tc_mid(1)


def _tc4_body(aggp_ref, hs_ref, dinv_ref, b_ref, out_ref):
    agg = jnp.sum(aggp_ref[...], axis=1, keepdims=True)
    out_ref[...] = ((agg + hs_ref[...]) * dinv_ref[...] + b_ref[...])[:N]


_tc4 = pl.pallas_call(
    _tc4_body, out_shape=jax.ShapeDtypeStruct((N, 1), jnp.float32))


def kernel(x, edge_index, W1, b1, g1, be1, W2, b2, g2, be2, W3, b3):
    ei = edge_index.astype(jnp.int32)
    src = jnp.concatenate([ei[0], jnp.zeros((EP - E,), jnp.int32)])
    dst = jnp.concatenate([ei[1], jnp.full((EP - E,), DUMMY, jnp.int32)])
    src3 = src.reshape(NW, K, CH)
    dst3 = dst.reshape(NW, K, CH)
    src2 = src.reshape(NW, EPW)
    dst2 = dst.reshape(NW, EPW)

    x_pad = jnp.pad(x, ((0, NP - N), (0, 0)))
    zeros32 = jnp.zeros((NP, 32), jnp.float32)

    h1 = _tc1a(x_pad, W1)
    degp = _degree(dst2).T
    hs1, dinv = _tc1b(h1, degp)

    aggp1 = _agg32(hs1, src3, dst3, zeros32)
    hs2 = _tc2(aggp1, hs1, dinv, b1.reshape(1, 32), g1.reshape(1, 32),
               be1.reshape(1, 32), W2)

    aggp2 = _agg32(hs2, src3, dst3, zeros32)
    hs3 = _tc3(aggp2, hs2, dinv, b2.reshape(1, 32), g2.reshape(1, 32),
               be2.reshape(1, 32), W3)

    aggp3 = _agg1(hs3.reshape(NP), src2, dst2).T
    return _tc4(aggp3, hs3, dinv, b3.reshape(1, 1))


# speedup vs baseline: 1.0335x; 1.0335x over previous
import functools
import math

import jax
import jax.numpy as jnp
from jax import lax
from jax.experimental import pallas as pl
from jax.experimental.pallas import tpu as pltpu
from jax.experimental.pallas import tpu_sc as plsc

N = 10000
NP = 10240
E = 320000
NC = 2
NS = 16
NW = NC * NS
CH = 128
K = 80
EPW = K * CH
EP = NW * EPW
RPT = NP // NS
L = 16
DUMMY = N
INV_SQRT1P = 1.0 / math.sqrt(1.0 + 1e-5)

_mesh = plsc.VectorSubcoreMesh(core_axis_name="c", subcore_axis_name="s")
_sc_params = pltpu.CompilerParams(use_tc_tiling_on_sc=False,
                                  needs_layout_passes=False)


NBUF = 8
LAG = NBUF // 2


def _make_agg(F):
    @functools.partial(
        pl.kernel,
        out_type=jax.ShapeDtypeStruct((NP, NC * F), jnp.float32),
        mesh=_mesh,
        compiler_params=_sc_params,
        scratch_types=[
            pltpu.VMEM((K, CH), jnp.int32),
            pltpu.VMEM((K, CH), jnp.int32),
            pltpu.VMEM((NBUF, 2, CH, F), jnp.float32),
            pltpu.VMEM_SHARED((NP, F), jnp.float32),
            pltpu.VMEM_SHARED((NP, F), jnp.float32),
        ] + [pltpu.SemaphoreType.DMA] * (2 * NBUF),
    )
    def agg(table_hbm, src_hbm, dst_hbm, zeros_hbm, out_hbm,
            src_v, dst_v, rows_v, acc_sh, table_sh, *sems):
        gsem = sems[:NBUF]
        ssem = sems[NBUF:]
        c = lax.axis_index("c")
        s = lax.axis_index("s")
        w = c * NS + s

        pltpu.sync_copy(src_hbm.at[w], src_v)
        pltpu.sync_copy(dst_hbm.at[w], dst_v)
        pltpu.sync_copy(table_hbm.at[pl.ds(s * RPT, RPT)],
                        table_sh.at[pl.ds(s * RPT, RPT)])
        pltpu.sync_copy(zeros_hbm.at[pl.ds(s * RPT, RPT)],
                        acc_sh.at[pl.ds(s * RPT, RPT)])
        plsc.subcore_barrier()

        K2 = K // 2

        def gather(j):
            b = j % NBUF
            for h in range(2):
                pltpu.async_copy(table_sh.at[src_v.at[2 * j + h]],
                                 rows_v.at[b, h], gsem[b])

        def gather_wait(j):
            b = j % NBUF
            for h in range(2):
                pltpu.make_async_copy(table_sh.at[src_v.at[2 * j + h]],
                                      rows_v.at[b, h], gsem[b]).wait()

        def scatter(j):
            b = j % NBUF
            for h in range(2):
                pltpu.async_copy(rows_v.at[b, h],
                                 acc_sh.at[dst_v.at[2 * j + h]],
                                 ssem[b], add=True)

        def scatter_wait(j):
            b = j % NBUF
            for h in range(2):
                pltpu.make_async_copy(rows_v.at[b, h],
                                      acc_sh.at[dst_v.at[2 * j + h]],
                                      ssem[b]).wait()

        for j in range(NBUF):
            gather(j)
        for j in range(K2):
            gather_wait(j)
            scatter(j)
            jj = j - LAG
            if jj >= 0 and jj + NBUF < K2:
                scatter_wait(jj)
                gather(jj + NBUF)
        for jj in range(max(0, K2 - NBUF), K2):
            scatter_wait(jj)

        plsc.subcore_barrier()
        pltpu.sync_copy(acc_sh.at[pl.ds(s * RPT, RPT)],
                        out_hbm.at[pl.ds(s * RPT, RPT), pl.ds(c * F, F)])

    return agg


_agg32 = _make_agg(32)


def _make_agg1(with_table):
    scratch = [
        pltpu.VMEM((EPW,), jnp.int32),
        pltpu.VMEM((NP,), jnp.float32),
    ]
    if with_table:
        scratch = [pltpu.VMEM((EPW,), jnp.int32)] + scratch
        scratch.append(pltpu.VMEM((NP,), jnp.float32))

    @functools.partial(
        pl.kernel,
        out_type=jax.ShapeDtypeStruct((NW, NP), jnp.float32),
        mesh=_mesh,
        compiler_params=_sc_params,
        scratch_types=scratch,
    )
    def agg1(*refs):
        if with_table:
            (table_hbm, src_hbm, dst_hbm, out_hbm,
             src_v, dst_v, acc_v, table_v) = refs
        else:
            dst_hbm, out_hbm, dst_v, acc_v = refs
        c = lax.axis_index("c")
        s = lax.axis_index("s")
        w = c * NS + s

        pltpu.sync_copy(dst_hbm.at[w], dst_v)
        if with_table:
            pltpu.sync_copy(src_hbm.at[w], src_v)
            pltpu.sync_copy(table_hbm, table_v)

        zeros16 = jnp.zeros((L,), jnp.float32)

        @plsc.parallel_loop(0, NP, step=L, unroll=8)
        def _(i):
            acc_v[pl.ds(i, L)] = zeros16

        ones16 = jnp.ones((L,), jnp.float32)

        @plsc.parallel_loop(0, EPW, step=L, unroll=8)
        def _(i):
            d16 = dst_v[pl.ds(i, L)]
            if with_table:
                s16 = src_v[pl.ds(i, L)]
                v16 = plsc.load_gather(table_v, [s16])
            else:
                v16 = ones16
            plsc.addupdate_scatter(acc_v, [d16], v16)

        pltpu.sync_copy(acc_v, out_hbm.at[w])

    return agg1


_agg1 = _make_agg1(True)
_degree = _make_agg1(False)


def _tc1a_body(x_ref, w1_ref, h_ref):
    h_ref[...] = jnp.dot(x_ref[...], w1_ref[...],
                         preferred_element_type=jnp.float32)


_tc1a = pl.pallas_call(
    _tc1a_body, out_shape=jax.ShapeDtypeStruct((NP, 32), jnp.float32))


def _tc1b_body(h_ref, degp_ref, hs_ref, dinv_ref):
    deg = jnp.sum(degp_ref[...], axis=0, keepdims=True) + 1.0
    dinv = lax.rsqrt(deg).T
    dinv_ref[...] = dinv
    hs_ref[...] = h_ref[...] * dinv


_tc1b = pl.pallas_call(
    _tc1b_body,
    out_shape=(jax.ShapeDtypeStruct((NP, 32), jnp.float32),
               jax.ShapeDtypeStruct((NP, 1), jnp.float32)),
)


def _make_tc_mid(Fout):
    def body(aggp_ref, hs_ref, dinv_ref, b_ref, g_ref, be_ref, w_ref, out_ref):
        dinv = dinv_ref[...]
        aggp = aggp_ref[...]
        agg = (aggp[:, :32] + aggp[:, 32:] + hs_ref[...]) * dinv + b_ref[...]
        hin = jnp.maximum(agg * INV_SQRT1P * g_ref[...] + be_ref[...], 0.0)
        h = jnp.dot(hin, w_ref[...], preferred_element_type=jnp.float32)
        out_ref[...] = h * dinv

    return pl.pallas_call(
        body, out_shape=jax.ShapeDtypeStruct((NP, Fout), jnp.float32))


_tc2 = _make_tc_mid(32)
_tc3 = _make_tc_mid(1)


def _tc4_body(aggp_ref, hs_ref, dinv_ref, b_ref, out_ref):
    agg = jnp.sum(aggp_ref[...], axis=0, keepdims=True).T
    out_ref[...] = ((agg + hs_ref[...]) * dinv_ref[...] + b_ref[...])[:N]


_tc4 = pl.pallas_call(
    _tc4_body, out_shape=jax.ShapeDtypeStruct((N, 1), jnp.float32))


def kernel(x, edge_index, W1, b1, g1, be1, W2, b2, g2, be2, W3, b3):
    ei = edge_index.astype(jnp.int32)
    src = jnp.concatenate([ei[0], jnp.zeros((EP - E,), jnp.int32)])
    dst = jnp.concatenate([ei[1], jnp.full((EP - E,), DUMMY, jnp.int32)])
    src3 = src.reshape(NW, K, CH)
    dst3 = dst.reshape(NW, K, CH)
    src2 = src.reshape(NW, EPW)
    dst2 = dst.reshape(NW, EPW)

    x_pad = jnp.pad(x, ((0, NP - N), (0, 0)))
    zeros32 = jnp.zeros((NP, 32), jnp.float32)

    h1 = _tc1a(x_pad, W1)
    degp = _degree(dst2)
    hs1, dinv = _tc1b(h1, degp)

    aggp1 = _agg32(hs1, src3, dst3, zeros32)
    hs2 = _tc2(aggp1, hs1, dinv, b1.reshape(1, 32), g1.reshape(1, 32),
               be1.reshape(1, 32), W2)

    aggp2 = _agg32(hs2, src3, dst3, zeros32)
    hs3 = _tc3(aggp2, hs2, dinv, b2.reshape(1, 32), g2.reshape(1, 32),
               be2.reshape(1, 32), W3)

    aggp3 = _agg1(hs3.reshape(NP), src2, dst2)
    return _tc4(aggp3, hs3, dinv, b3.reshape(1, 1))


# speedup vs baseline: 1.0535x; 1.0194x over previous
import functools
import math

import jax
import jax.numpy as jnp
from jax import lax
from jax.experimental import pallas as pl
from jax.experimental.pallas import tpu as pltpu
from jax.experimental.pallas import tpu_sc as plsc

N = 10000
NP = 10240
E = 320000
NC = 2
NS = 16
NW = NC * NS
CH = 128
K = 80
EPW = K * CH
EP = NW * EPW
RPT = NP // NS
L = 16
DUMMY = N
INV_SQRT1P = 1.0 / math.sqrt(1.0 + 1e-5)

_mesh = plsc.VectorSubcoreMesh(core_axis_name="c", subcore_axis_name="s")
_sc_params = pltpu.CompilerParams(use_tc_tiling_on_sc=False,
                                  needs_layout_passes=False)


NBUF = 8
LAG = NBUF // 2


def _make_agg(F):
    @functools.partial(
        pl.kernel,
        out_type=jax.ShapeDtypeStruct((NP, NC * F), jnp.float32),
        mesh=_mesh,
        compiler_params=_sc_params,
        scratch_types=[
            pltpu.VMEM((K, CH), jnp.int32),
            pltpu.VMEM((K, CH), jnp.int32),
            pltpu.VMEM((NBUF, 2, CH, F), jnp.float32),
            pltpu.VMEM_SHARED((NP, F), jnp.float32),
            pltpu.VMEM_SHARED((NP, F), jnp.float32),
        ] + [pltpu.SemaphoreType.DMA] * (2 * NBUF),
    )
    def agg(table_hbm, src_hbm, dst_hbm, zeros_hbm, out_hbm,
            src_v, dst_v, rows_v, acc_sh, table_sh, *sems):
        gsem = sems[:NBUF]
        ssem = sems[NBUF:]
        c = lax.axis_index("c")
        s = lax.axis_index("s")
        w = c * NS + s

        pltpu.sync_copy(src_hbm.at[w], src_v)
        pltpu.sync_copy(dst_hbm.at[w], dst_v)
        pltpu.sync_copy(table_hbm.at[pl.ds(s * RPT, RPT)],
                        table_sh.at[pl.ds(s * RPT, RPT)])

        @pl.when(c == 0)
        def _():
            pltpu.sync_copy(table_hbm.at[pl.ds(s * RPT, RPT)],
                            acc_sh.at[pl.ds(s * RPT, RPT)])

        @pl.when(c != 0)
        def _():
            pltpu.sync_copy(zeros_hbm.at[pl.ds(s * RPT, RPT)],
                            acc_sh.at[pl.ds(s * RPT, RPT)])

        plsc.subcore_barrier()

        K2 = K // 2

        def gather(j):
            b = j % NBUF
            for h in range(2):
                pltpu.async_copy(table_sh.at[src_v.at[2 * j + h]],
                                 rows_v.at[b, h], gsem[b])

        def gather_wait(j):
            b = j % NBUF
            for h in range(2):
                pltpu.make_async_copy(table_sh.at[src_v.at[2 * j + h]],
                                      rows_v.at[b, h], gsem[b]).wait()

        def scatter(j):
            b = j % NBUF
            for h in range(2):
                pltpu.async_copy(rows_v.at[b, h],
                                 acc_sh.at[dst_v.at[2 * j + h]],
                                 ssem[b], add=True)

        def scatter_wait(j):
            b = j % NBUF
            for h in range(2):
                pltpu.make_async_copy(rows_v.at[b, h],
                                      acc_sh.at[dst_v.at[2 * j + h]],
                                      ssem[b]).wait()

        for j in range(NBUF):
            gather(j)
        for j in range(K2):
            gather_wait(j)
            scatter(j)
            jj = j - LAG
            if jj >= 0 and jj + NBUF < K2:
                scatter_wait(jj)
                gather(jj + NBUF)
        for jj in range(max(0, K2 - NBUF), K2):
            scatter_wait(jj)

        plsc.subcore_barrier()
        pltpu.sync_copy(acc_sh.at[pl.ds(s * RPT, RPT)],
                        out_hbm.at[pl.ds(s * RPT, RPT), pl.ds(c * F, F)])

    return agg


_agg32 = _make_agg(32)


def _make_agg1(with_table):
    scratch = [
        pltpu.VMEM((EPW,), jnp.int32),
        pltpu.VMEM((NP,), jnp.float32),
    ]
    if with_table:
        scratch = [pltpu.VMEM((EPW,), jnp.int32)] + scratch
        scratch.append(pltpu.VMEM((NP,), jnp.float32))

    @functools.partial(
        pl.kernel,
        out_type=jax.ShapeDtypeStruct((NW, NP), jnp.float32),
        mesh=_mesh,
        compiler_params=_sc_params,
        scratch_types=scratch,
    )
    def agg1(*refs):
        if with_table:
            (table_hbm, src_hbm, dst_hbm, out_hbm,
             src_v, dst_v, acc_v, table_v) = refs
        else:
            dst_hbm, out_hbm, dst_v, acc_v = refs
        c = lax.axis_index("c")
        s = lax.axis_index("s")
        w = c * NS + s

        pltpu.sync_copy(dst_hbm.at[w], dst_v)
        if with_table:
            pltpu.sync_copy(src_hbm.at[w], src_v)
            pltpu.sync_copy(table_hbm, table_v)

        zeros16 = jnp.zeros((L,), jnp.float32)

        if with_table:
            @pl.when(w == 0)
            def _():
                pltpu.sync_copy(table_hbm, acc_v)

            @pl.when(w != 0)
            def _():
                @plsc.parallel_loop(0, NP, step=L, unroll=8)
                def _(i):
                    acc_v[pl.ds(i, L)] = zeros16
        else:
            @plsc.parallel_loop(0, NP, step=L, unroll=8)
            def _(i):
                acc_v[pl.ds(i, L)] = zeros16

        ones16 = jnp.ones((L,), jnp.float32)

        @plsc.parallel_loop(0, EPW, step=L, unroll=8)
        def _(i):
            d16 = dst_v[pl.ds(i, L)]
            if with_table:
                s16 = src_v[pl.ds(i, L)]
                v16 = plsc.load_gather(table_v, [s16])
            else:
                v16 = ones16
            plsc.addupdate_scatter(acc_v, [d16], v16)

        pltpu.sync_copy(acc_v, out_hbm.at[w])

    return agg1


_agg1 = _make_agg1(True)
_degree = _make_agg1(False)


def _tc1a_body(x_ref, w1_ref, h_ref):
    h_ref[pl.ds(0, N), :] = jnp.dot(x_ref[...], w1_ref[...],
                                    preferred_element_type=jnp.float32)


_tc1a = pl.pallas_call(
    _tc1a_body, out_shape=jax.ShapeDtypeStruct((NP, 32), jnp.float32))


def _tc1b_body(h_ref, degp_ref, hs_ref, dinv_ref):
    deg = jnp.sum(degp_ref[...], axis=0, keepdims=True) + 1.0
    dinv = lax.rsqrt(deg).T
    dinv_ref[...] = dinv
    hs_ref[...] = h_ref[...] * dinv


_tc1b = pl.pallas_call(
    _tc1b_body,
    out_shape=(jax.ShapeDtypeStruct((NP, 32), jnp.float32),
               jax.ShapeDtypeStruct((NP, 1), jnp.float32)),
)


def _make_tc_mid(Fout):
    def body(aggp_ref, dinv_ref, b_ref, g_ref, be_ref, w_ref, out_ref):
        dinv = dinv_ref[...]
        aggp = aggp_ref[...]
        agg = (aggp[:, :32] + aggp[:, 32:]) * dinv + b_ref[...]
        hin = jnp.maximum(agg * INV_SQRT1P * g_ref[...] + be_ref[...], 0.0)
        h = jnp.dot(hin, w_ref[...], preferred_element_type=jnp.float32)
        out_ref[...] = h * dinv

    return pl.pallas_call(
        body, out_shape=jax.ShapeDtypeStruct((NP, Fout), jnp.float32))


_tc2 = _make_tc_mid(32)
_tc3 = _make_tc_mid(1)


def _tc4_body(aggp_ref, dinv_ref, b_ref, out_ref):
    agg = jnp.sum(aggp_ref[...], axis=0, keepdims=True).T
    out_ref[...] = (agg * dinv_ref[...] + b_ref[...])[:N]


_tc4 = pl.pallas_call(
    _tc4_body, out_shape=jax.ShapeDtypeStruct((N, 1), jnp.float32))


def kernel(x, edge_index, W1, b1, g1, be1, W2, b2, g2, be2, W3, b3):
    ei = edge_index.astype(jnp.int32)
    src = jnp.concatenate([ei[0], jnp.zeros((EP - E,), jnp.int32)])
    dst = jnp.concatenate([ei[1], jnp.full((EP - E,), DUMMY, jnp.int32)])
    src3 = src.reshape(NW, K, CH)
    dst3 = dst.reshape(NW, K, CH)
    src2 = src.reshape(NW, EPW)
    dst2 = dst.reshape(NW, EPW)

    zeros32 = jnp.zeros((NP, 32), jnp.float32)

    h1 = _tc1a(x, W1)
    degp = _degree(dst2)
    hs1, dinv = _tc1b(h1, degp)

    aggp1 = _agg32(hs1, src3, dst3, zeros32)
    hs2 = _tc2(aggp1, dinv, b1.reshape(1, 32), g1.reshape(1, 32),
               be1.reshape(1, 32), W2)

    aggp2 = _agg32(hs2, src3, dst3, zeros32)
    hs3 = _tc3(aggp2, dinv, b2.reshape(1, 32), g2.reshape(1, 32),
               be2.reshape(1, 32), W3)

    aggp3 = _agg1(hs3.reshape(NP), src2, dst2)
    return _tc4(aggp3, dinv, b3.reshape(1, 1))
